# Initial kernel scaffold; baseline (speedup 1.0000x reference)
import jax
import jax.numpy as jnp
from jax.experimental import pallas as pl


def kernel(DPTD_name_1, DPTD_name_2, emb_table, W1, b1, W2, b2, p):
    raise NotImplementedError("write your pallas kernel here")


# speedup vs baseline: 1.4752x; 1.4752x over previous
import functools

import jax
import jax.numpy as jnp
from jax import lax
from jax.experimental import pallas as pl
from jax.experimental.pallas import tpu as pltpu
from jax.experimental.pallas import tpu_sc as plsc

_VOCAB = 1000
_VPAD = 1024
_D = 10
_B = 16384
_NC, _NS, _L = 2, 16, 16
_NW = _NC * _NS
_BPW = _B // _NW
_GROUPS = _BPW // _L
_TFLAT = _D * _VPAD


def _tables_body(embp_ref, w1_ref, b1_ref, w2_ref, b2_ref,
                 ptabt_ref, etabt_ref):
    pt = lax.dot_general(w1_ref[...], embp_ref[...], (((1,), (1,)), ((), ())),
                         preferred_element_type=jnp.float32) + b1_ref[...][:, None]
    tt = jnp.tanh(pt)
    st = lax.dot_general(w2_ref[...], tt, (((1,), (0,)), ((), ())),
                         preferred_element_type=jnp.float32) + b2_ref[...][:, None]
    ptabt_ref[...] = pt
    etabt_ref[...] = tt + st


_tables = pl.pallas_call(
    _tables_body,
    out_shape=[jax.ShapeDtypeStruct((_D, _VPAD), jnp.float32),
               jax.ShapeDtypeStruct((_D, _VPAD), jnp.float32)],
)


def _rsqrt_nr(x):
    i = plsc.bitcast(x, jnp.int32)
    i = jnp.int32(0x5F3759DF) - lax.shift_right_logical(i, 1)
    y = plsc.bitcast(i, jnp.float32)
    for _ in range(3):
        y = y * (1.5 - 0.5 * x * y * y)
    return y


_sc_mesh = plsc.VectorSubcoreMesh(core_axis_name="c", subcore_axis_name="s")


@functools.partial(
    pl.kernel,
    mesh=_sc_mesh,
    compiler_params=pltpu.CompilerParams(needs_layout_passes=False),
    out_type=[jax.ShapeDtypeStruct((_B * _D,), jnp.float32),
              jax.ShapeDtypeStruct((_B * _D,), jnp.float32),
              jax.ShapeDtypeStruct((_B,), jnp.float32),
              jax.ShapeDtypeStruct((_NW, _L), jnp.float32)],
    scratch_types=[
        pltpu.VMEM((_BPW,), jnp.int32),
        pltpu.VMEM((_BPW,), jnp.int32),
        pltpu.VMEM((_TFLAT,), jnp.float32),
        pltpu.VMEM((_TFLAT,), jnp.float32),
        pltpu.VMEM((_BPW * _D,), jnp.float32),
        pltpu.VMEM((_BPW * _D,), jnp.float32),
        pltpu.VMEM((_BPW,), jnp.float32),
        pltpu.VMEM((_L,), jnp.float32),
    ],
)
def _sc_main(ptabt_hbm, etabt_hbm, idx1_hbm, idx2_hbm,
             x1_hbm, emb_hbm, cos_hbm, parts_hbm,
             idx1_v, idx2_v, ptabt_v, etabt_v,
             out1_v, out2_v, cos_v, acc_v):
    wid = lax.axis_index("s") * _NC + lax.axis_index("c")
    base = wid * _BPW

    pltpu.sync_copy(idx1_hbm.at[pl.ds(base, _BPW)], idx1_v)
    pltpu.sync_copy(idx2_hbm.at[pl.ds(base, _BPW)], idx2_v)
    pltpu.sync_copy(ptabt_hbm, ptabt_v)
    pltpu.sync_copy(etabt_hbm, etabt_v)

    def group(g, dist_acc):
        o = g * _L
        i1v = idx1_v[pl.ds(o, _L)]
        i2v = idx2_v[pl.ds(o, _L)]
        rowbase = o * _D + lax.iota(jnp.int32, _L) * _D
        dotv = jnp.zeros((_L,), jnp.float32)
        n1v = jnp.zeros((_L,), jnp.float32)
        n2v = jnp.zeros((_L,), jnp.float32)
        for j in range(_D):
            r1 = plsc.load_gather(ptabt_v, [i1v + jnp.int32(j * _VPAD)])
            r2 = plsc.load_gather(etabt_v, [i2v + jnp.int32(j * _VPAD)])
            plsc.store_scatter(out1_v, [rowbase + jnp.int32(j)], r1)
            plsc.store_scatter(out2_v, [rowbase + jnp.int32(j)], r2)
            dotv = dotv + r1 * r2
            n1v = n1v + r1 * r1
            n2v = n2v + r2 * r2
        q = jnp.maximum(n1v * n2v, jnp.float32(1e-16))
        cos_v[pl.ds(o, _L)] = dotv * _rsqrt_nr(q)
        return dist_acc + (n1v + n2v - 2.0 * dotv)

    dist_vec = lax.fori_loop(0, _GROUPS, group,
                             jnp.zeros((_L,), jnp.float32))
    acc_v[...] = dist_vec

    pltpu.sync_copy(out1_v, x1_hbm.at[pl.ds(base * _D, _BPW * _D)])
    pltpu.sync_copy(out2_v, emb_hbm.at[pl.ds(base * _D, _BPW * _D)])
    pltpu.sync_copy(cos_v, cos_hbm.at[pl.ds(base, _BPW)])
    pltpu.sync_copy(acc_v, parts_hbm.at[wid])


def _finish_body(parts_ref, p_ref, cos_ref, sims_ref):
    dist = jnp.sqrt(jnp.maximum(jnp.sum(parts_ref[...]), 0.0))
    sims_ref[...] = p_ref[0] * cos_ref[...] + p_ref[1] * dist


_finish = pl.pallas_call(
    _finish_body,
    in_specs=[pl.BlockSpec(memory_space=pltpu.VMEM),
              pl.BlockSpec(memory_space=pltpu.SMEM),
              pl.BlockSpec(memory_space=pltpu.VMEM)],
    out_shape=jax.ShapeDtypeStruct((128, 128), jnp.float32),
)


def kernel(DPTD_name_1, DPTD_name_2, emb_table, W1, b1, W2, b2, p):
    idx1 = DPTD_name_1.astype(jnp.int32)
    idx2 = DPTD_name_2.astype(jnp.int32)
    emb_pad = jnp.pad(emb_table, ((0, _VPAD - _VOCAB), (0, 0)))
    ptabt, etabt = _tables(emb_pad, W1, b1, W2, b2)
    x1f, embf, cos, parts = _sc_main(
        ptabt.reshape(_TFLAT), etabt.reshape(_TFLAT), idx1, idx2)
    sims = _finish(parts, p, cos.reshape(128, 128)).reshape(_B)
    return (sims, x1f.reshape(_B, _D), embf.reshape(_B, _D))
